# initial kernel scaffold (unmeasured)
import jax
import jax.numpy as jnp
from jax import lax
from jax.experimental import pallas as pl
from jax.experimental.pallas import tpu as pltpu

N_DEV = 4
B, SEQ, H, D = 2, 256, 8, 64
SG = SEQ * N_DEV
ROWS, COLS = B * SEQ, H * D


def _ring_allgather_kv(K2, V2):

    def body(k_ref, v_ref, ko_ref, vo_ref, ck, cv, ksend, krecv, vsend, vrecv):
        my = lax.axis_index("i")
        left = lax.rem(my + N_DEV - 1, N_DEV)
        right = lax.rem(my + 1, N_DEV)

        barrier = pltpu.get_barrier_semaphore()
        for nbr in (left, right):
            pl.semaphore_signal(
                barrier, inc=1,
                device_id=(nbr,), device_id_type=pl.DeviceIdType.MESH,
            )
        pl.semaphore_wait(barrier, 2)

        kb = k_ref[...].astype(jnp.bfloat16)
        vb = v_ref[...].astype(jnp.bfloat16)
        ck[0] = kb
        cv[0] = vb
        for b in range(B):
            ko_ref[pl.ds(b * SG + my * SEQ, SEQ), :] = kb[b * SEQ:(b + 1) * SEQ, :]
            vo_ref[pl.ds(b * SG + my * SEQ, SEQ), :] = vb[b * SEQ:(b + 1) * SEQ, :]

        for h in range(N_DEV - 1):
            rk = pltpu.make_async_remote_copy(
                src_ref=ck.at[h], dst_ref=ck.at[h + 1],
                send_sem=ksend.at[h], recv_sem=krecv.at[h],
                device_id=(right,), device_id_type=pl.DeviceIdType.MESH,
            )
            rv = pltpu.make_async_remote_copy(
                src_ref=cv.at[h], dst_ref=cv.at[h + 1],
                send_sem=vsend.at[h], recv_sem=vrecv.at[h],
                device_id=(left,), device_id_type=pl.DeviceIdType.MESH,
            )
            rk.start()
            rv.start()
            rk.wait()
            rv.wait()

            ko_origin = lax.rem(my + N_DEV - 1 - h, N_DEV)
            vo_origin = lax.rem(my + 1 + h, N_DEV)
            for b in range(B):
                ko_ref[pl.ds(b * SG + ko_origin * SEQ, SEQ), :] = (
                    ck[h + 1, b * SEQ:(b + 1) * SEQ, :]
                )
                vo_ref[pl.ds(b * SG + vo_origin * SEQ, SEQ), :] = (
                    cv[h + 1, b * SEQ:(b + 1) * SEQ, :]
                )

    out2d = jax.ShapeDtypeStruct((N_DEV * ROWS, COLS), jnp.bfloat16)
    return pl.pallas_call(
        body,
        out_shape=(out2d, out2d),
        in_specs=[
            pl.BlockSpec(memory_space=pltpu.VMEM),
            pl.BlockSpec(memory_space=pltpu.VMEM),
        ],
        out_specs=(
            pl.BlockSpec(memory_space=pltpu.VMEM),
            pl.BlockSpec(memory_space=pltpu.VMEM),
        ),
        scratch_shapes=[
            pltpu.VMEM((N_DEV, ROWS, COLS), jnp.bfloat16),
            pltpu.VMEM((N_DEV, ROWS, COLS), jnp.bfloat16),
            pltpu.SemaphoreType.DMA((N_DEV - 1,)),
            pltpu.SemaphoreType.DMA((N_DEV - 1,)),
            pltpu.SemaphoreType.DMA((N_DEV - 1,)),
            pltpu.SemaphoreType.DMA((N_DEV - 1,)),
        ],
        compiler_params=pltpu.CompilerParams(collective_id=0),
    )(K2, V2)


def _attention(Q4, K4, V4):

    def body(q_ref, k_ref, v_ref, o_ref):
        q = q_ref[...].astype(jnp.bfloat16)
        k = k_ref[...]
        v = v_ref[...]
        s = lax.dot_general(
            q, k, (((1,), (1,)), ((), ())),
            preferred_element_type=jnp.float32,
        ) * (D ** -0.5)
        m = jnp.max(s, axis=-1, keepdims=True)
        p = jnp.exp(s - m)
        l = jnp.sum(p, axis=-1, keepdims=True)
        o = lax.dot_general(
            p.astype(jnp.bfloat16), v, (((1,), (0,)), ((), ())),
            preferred_element_type=jnp.float32,
        )
        o_ref[...] = o / l

    return pl.pallas_call(
        body,
        grid=(B, H),
        in_specs=[
            pl.BlockSpec((None, SEQ, None, D), lambda b, h: (b, 0, h, 0)),
            pl.BlockSpec((None, SG, None, D), lambda b, h: (b, 0, h, 0)),
            pl.BlockSpec((None, SG, None, D), lambda b, h: (b, 0, h, 0)),
        ],
        out_specs=pl.BlockSpec((None, SEQ, None, D), lambda b, h: (b, 0, h, 0)),
        out_shape=jax.ShapeDtypeStruct((B, SEQ, H, D), jnp.float32),
    )(Q4, K4, V4)


def kernel(Q, K, V):
    K2 = K.reshape(ROWS, COLS)
    V2 = V.reshape(ROWS, COLS)
    KO2, VO2 = _ring_allgather_kv(K2, V2)
    K4 = KO2.reshape(B, SG, H, D)
    V4 = VO2.reshape(B, SG, H, D)
    return _attention(Q, K4, V4)


# baseline (device time: 42980 ns/iter reference)
import jax
import jax.numpy as jnp
from jax import lax
from jax.experimental import pallas as pl
from jax.experimental.pallas import tpu as pltpu

N_DEV = 4
B, SEQ, H, D = 2, 256, 8, 64
SG = SEQ * N_DEV
ROWS, COLS = B * SEQ, H * D


def _ring_allgather_kv(K2, V2):

    def body(k_ref, v_ref, ko_ref, vo_ref, ck, cv, ksend, krecv, vsend, vrecv):
        my = lax.axis_index("i")
        left = lax.rem(my + N_DEV - 1, N_DEV)
        right = lax.rem(my + 1, N_DEV)

        barrier = pltpu.get_barrier_semaphore()
        for nbr in (left, right):
            pl.semaphore_signal(
                barrier, inc=1,
                device_id=(nbr,), device_id_type=pl.DeviceIdType.MESH,
            )
        pl.semaphore_wait(barrier, 2)

        kb = k_ref[...].astype(jnp.bfloat16)
        vb = v_ref[...].astype(jnp.bfloat16)
        ck[0] = kb
        cv[0] = vb
        for b in range(B):
            ko_ref[pl.ds(b * SG + my * SEQ, SEQ), :] = kb[b * SEQ:(b + 1) * SEQ, :]
            vo_ref[pl.ds(b * SG + my * SEQ, SEQ), :] = vb[b * SEQ:(b + 1) * SEQ, :]

        for h in range(N_DEV - 1):
            rk = pltpu.make_async_remote_copy(
                src_ref=ck.at[h], dst_ref=ck.at[h + 1],
                send_sem=ksend.at[h], recv_sem=krecv.at[h],
                device_id=(right,), device_id_type=pl.DeviceIdType.MESH,
            )
            rv = pltpu.make_async_remote_copy(
                src_ref=cv.at[h], dst_ref=cv.at[h + 1],
                send_sem=vsend.at[h], recv_sem=vrecv.at[h],
                device_id=(left,), device_id_type=pl.DeviceIdType.MESH,
            )
            rk.start()
            rv.start()
            rk.wait()
            rv.wait()

            ko_origin = lax.rem(my + N_DEV - 1 - h, N_DEV)
            vo_origin = lax.rem(my + 1 + h, N_DEV)
            for b in range(B):
                ko_ref[pl.ds(b * SG + ko_origin * SEQ, SEQ), :] = (
                    ck[h + 1, b * SEQ:(b + 1) * SEQ, :]
                )
                vo_ref[pl.ds(b * SG + vo_origin * SEQ, SEQ), :] = (
                    cv[h + 1, b * SEQ:(b + 1) * SEQ, :]
                )

    out2d = jax.ShapeDtypeStruct((N_DEV * ROWS, COLS), jnp.bfloat16)
    return pl.pallas_call(
        body,
        out_shape=(out2d, out2d),
        in_specs=[
            pl.BlockSpec(memory_space=pltpu.VMEM),
            pl.BlockSpec(memory_space=pltpu.VMEM),
        ],
        out_specs=(
            pl.BlockSpec(memory_space=pltpu.VMEM),
            pl.BlockSpec(memory_space=pltpu.VMEM),
        ),
        scratch_shapes=[
            pltpu.VMEM((N_DEV, ROWS, COLS), jnp.bfloat16),
            pltpu.VMEM((N_DEV, ROWS, COLS), jnp.bfloat16),
            pltpu.SemaphoreType.DMA((N_DEV - 1,)),
            pltpu.SemaphoreType.DMA((N_DEV - 1,)),
            pltpu.SemaphoreType.DMA((N_DEV - 1,)),
            pltpu.SemaphoreType.DMA((N_DEV - 1,)),
        ],
        compiler_params=pltpu.CompilerParams(collective_id=0),
    )(K2, V2)


def _attention(Q2, K2g, V2g):

    def body(q_ref, k_ref, v_ref, o_ref):
        q = q_ref[...].astype(jnp.bfloat16)
        k = k_ref[...]
        v = v_ref[...]
        outs = []
        for h in range(H):
            sl = slice(h * D, (h + 1) * D)
            s = lax.dot_general(
                q[:, sl], k[:, sl], (((1,), (1,)), ((), ())),
                preferred_element_type=jnp.float32,
            ) * (D ** -0.5)
            m = jnp.max(s, axis=-1, keepdims=True)
            p = jnp.exp(s - m)
            l = jnp.sum(p, axis=-1, keepdims=True)
            o = lax.dot_general(
                p.astype(jnp.bfloat16), v[:, sl], (((1,), (0,)), ((), ())),
                preferred_element_type=jnp.float32,
            )
            outs.append(o / l)
        o_ref[...] = jnp.concatenate(outs, axis=1)

    return pl.pallas_call(
        body,
        grid=(B,),
        in_specs=[
            pl.BlockSpec((SEQ, COLS), lambda b: (b, 0)),
            pl.BlockSpec((SG, COLS), lambda b: (b, 0)),
            pl.BlockSpec((SG, COLS), lambda b: (b, 0)),
        ],
        out_specs=pl.BlockSpec((SEQ, COLS), lambda b: (b, 0)),
        out_shape=jax.ShapeDtypeStruct((ROWS, COLS), jnp.float32),
    )(Q2, K2g, V2g)


def kernel(Q, K, V):
    Q2 = Q.reshape(ROWS, COLS)
    K2 = K.reshape(ROWS, COLS)
    V2 = V.reshape(ROWS, COLS)
    KO2, VO2 = _ring_allgather_kv(K2, V2)
    O2 = _attention(Q2, KO2, VO2)
    return O2.reshape(B, SEQ, H, D)


# device time: 41495 ns/iter; 1.0358x vs baseline; 1.0358x over previous
import jax
import jax.numpy as jnp
from jax import lax
from jax.experimental import pallas as pl
from jax.experimental.pallas import tpu as pltpu

N_DEV = 4
B, SEQ, H, D = 2, 256, 8, 64
SG = SEQ * N_DEV
ROWS, COLS = B * SEQ, H * D
HH = H // 2
HC = HH * D
SCALE = D ** -0.5


def _fused_ag_attention(Q2, K2, V2):

    def body(q_ref, k_ref, v_ref, o_ref,
             cw, ccw, cw_send, cw_recv, ccw_send, ccw_recv):
        my = lax.axis_index("i")
        left = lax.rem(my + N_DEV - 1, N_DEV)
        right = lax.rem(my + 1, N_DEV)

        barrier = pltpu.get_barrier_semaphore()
        for nbr in (left, right):
            pl.semaphore_signal(
                barrier, inc=1,
                device_id=(nbr,), device_id_type=pl.DeviceIdType.MESH,
            )
        pl.semaphore_wait(barrier, 2)

        q = q_ref[...].astype(jnp.bfloat16)
        kb = k_ref[...].astype(jnp.bfloat16)
        vb = v_ref[...].astype(jnp.bfloat16)
        cw[0, :ROWS, :] = kb[:, :HC]
        cw[0, ROWS:, :] = vb[:, :HC]
        ccw[0, :ROWS, :] = kb[:, HC:]
        ccw[0, ROWS:, :] = vb[:, HC:]

        rdmas = []
        for hop in range(N_DEV - 1):
            rcw = pltpu.make_async_remote_copy(
                src_ref=cw.at[hop], dst_ref=cw.at[hop + 1],
                send_sem=cw_send.at[hop], recv_sem=cw_recv.at[hop],
                device_id=(right,), device_id_type=pl.DeviceIdType.MESH,
            )
            rccw = pltpu.make_async_remote_copy(
                src_ref=ccw.at[hop], dst_ref=ccw.at[hop + 1],
                send_sem=ccw_send.at[hop], recv_sem=ccw_recv.at[hop],
                device_id=(left,), device_id_type=pl.DeviceIdType.MESH,
            )
            rdmas.append((rcw, rccw))

        state = {}

        def process(kchunk, vchunk, head_base):
            for b in range(B):
                rs = slice(b * SEQ, (b + 1) * SEQ)
                for hh in range(HH):
                    h = head_base + hh
                    cs = slice(hh * D, (hh + 1) * D)
                    s = lax.dot_general(
                        q[rs, h * D:(h + 1) * D], kchunk[rs, cs],
                        (((1,), (1,)), ((), ())),
                        preferred_element_type=jnp.float32,
                    ) * SCALE
                    mj = jnp.max(s, axis=-1, keepdims=True)
                    if (b, h) not in state:
                        p = jnp.exp(s - mj)
                        state[(b, h)] = (
                            mj,
                            jnp.sum(p, axis=-1, keepdims=True),
                            lax.dot_general(
                                p.astype(jnp.bfloat16), vchunk[rs, cs],
                                (((1,), (0,)), ((), ())),
                                preferred_element_type=jnp.float32,
                            ),
                        )
                    else:
                        m0, l0, a0 = state[(b, h)]
                        mn = jnp.maximum(m0, mj)
                        corr = jnp.exp(m0 - mn)
                        p = jnp.exp(s - mn)
                        state[(b, h)] = (
                            mn,
                            l0 * corr + jnp.sum(p, axis=-1, keepdims=True),
                            a0 * corr + lax.dot_general(
                                p.astype(jnp.bfloat16), vchunk[rs, cs],
                                (((1,), (0,)), ((), ())),
                                preferred_element_type=jnp.float32,
                            ),
                        )

        rdmas[0][0].start()
        rdmas[0][1].start()
        process(kb[:, :HC], vb[:, :HC], 0)
        process(kb[:, HC:], vb[:, HC:], HH)

        for hop in range(N_DEV - 1):
            rcw, rccw = rdmas[hop]
            rcw.wait_recv()
            rccw.wait_recv()
            if hop < N_DEV - 2:
                rdmas[hop + 1][0].start()
                rdmas[hop + 1][1].start()
            ch_cw = cw[hop + 1]
            ch_ccw = ccw[hop + 1]
            process(ch_cw[:ROWS, :], ch_cw[ROWS:, :], 0)
            process(ch_ccw[:ROWS, :], ch_ccw[ROWS:, :], HH)

        for rcw, rccw in rdmas:
            rcw.wait_send()
            rccw.wait_send()

        for b in range(B):
            row = jnp.concatenate(
                [state[(b, h)][2] / state[(b, h)][1] for h in range(H)],
                axis=1,
            )
            o_ref[b * SEQ:(b + 1) * SEQ, :] = row

    comm = pltpu.VMEM((N_DEV, 2 * ROWS, HC), jnp.bfloat16)
    return pl.pallas_call(
        body,
        out_shape=jax.ShapeDtypeStruct((ROWS, COLS), jnp.float32),
        in_specs=[
            pl.BlockSpec(memory_space=pltpu.VMEM),
            pl.BlockSpec(memory_space=pltpu.VMEM),
            pl.BlockSpec(memory_space=pltpu.VMEM),
        ],
        out_specs=pl.BlockSpec(memory_space=pltpu.VMEM),
        scratch_shapes=[
            comm,
            comm,
            pltpu.SemaphoreType.DMA((N_DEV - 1,)),
            pltpu.SemaphoreType.DMA((N_DEV - 1,)),
            pltpu.SemaphoreType.DMA((N_DEV - 1,)),
            pltpu.SemaphoreType.DMA((N_DEV - 1,)),
        ],
        compiler_params=pltpu.CompilerParams(collective_id=0),
    )(Q2, K2, V2)


def kernel(Q, K, V):
    Q2 = Q.reshape(ROWS, COLS)
    K2 = K.reshape(ROWS, COLS)
    V2 = V.reshape(ROWS, COLS)
    O2 = _fused_ag_attention(Q2, K2, V2)
    return O2.reshape(B, SEQ, H, D)


# device time: 32360 ns/iter; 1.3282x vs baseline; 1.2823x over previous
import os

import jax
import jax.numpy as jnp
from jax import lax
from jax.experimental import pallas as pl
from jax.experimental.pallas import tpu as pltpu

_MODE = os.environ.get("KERNEL_MODE", "full")
_SCOPES = os.environ.get("KERNEL_SCOPES", "") == "1"


def _sc(name):
    import contextlib

    return jax.named_scope(name) if _SCOPES else contextlib.nullcontext()

N_DEV = 4
B, SEQ, H, D = 2, 256, 8, 64
SG = SEQ * N_DEV
ROWS, COLS = B * SEQ, H * D
HH = H // 2
HC = HH * D
NSUB = B
SUBR = 2 * SEQ
SCALE = D ** -0.5


def _fused_ag_attention(Q2, K2, V2):

    def body(q_ref, k_ref, v_ref, o_ref,
             cwb, ccwb, cw_send, cw_recv, ccw_send, ccw_recv,
             dcw_send, dcw_recv, dccw_send, dccw_recv):
        my = lax.axis_index("i")
        left = lax.rem(my + N_DEV - 1, N_DEV)
        right = lax.rem(my + 1, N_DEV)

        q = q_ref[...]
        kb = k_ref[...]
        vb = v_ref[...]
        for b in range(B):
            rs = slice(b * SEQ, (b + 1) * SEQ)
            cwb[0, pl.ds(2 * b * SEQ, SEQ), :] = kb[rs, :HC]
            cwb[0, pl.ds((2 * b + 1) * SEQ, SEQ), :] = vb[rs, :HC]
            ccwb[0, pl.ds(2 * b * SEQ, SEQ), :] = kb[rs, HC:]
            ccwb[0, pl.ds((2 * b + 1) * SEQ, SEQ), :] = vb[rs, HC:]

        barrier = pltpu.get_barrier_semaphore()
        for nbr in (left, right):
            pl.semaphore_signal(
                barrier, inc=1,
                device_id=(nbr,), device_id_type=pl.DeviceIdType.MESH,
            )
        pl.semaphore_wait(barrier, 2)

        def make(buf, send, recv, src_slot, dst_slot, stage, sub, dev):
            return pltpu.make_async_remote_copy(
                src_ref=buf.at[src_slot, pl.ds(sub * SUBR, SUBR)],
                dst_ref=buf.at[dst_slot, pl.ds(sub * SUBR, SUBR)],
                send_sem=send.at[stage, sub], recv_sem=recv.at[stage, sub],
                device_id=(dev,), device_id_type=pl.DeviceIdType.MESH,
            )

        hop1_cw = [make(cwb, cw_send, cw_recv, 0, 1, 0, s, right)
                   for s in range(NSUB)]
        fwd_cw = [make(cwb, cw_send, cw_recv, 1, 2, 1, s, right)
                  for s in range(NSUB)]
        hop1_ccw = [make(ccwb, ccw_send, ccw_recv, 0, 1, 0, s, left)
                    for s in range(NSUB)]
        fwd_ccw = [make(ccwb, ccw_send, ccw_recv, 1, 2, 1, s, left)
                   for s in range(NSUB)]

        _DIR_ROWS = [0, 2 * SEQ, SEQ, 3 * SEQ]

        def make_dir(buf, send, recv, i, dev):
            return pltpu.make_async_remote_copy(
                src_ref=buf.at[0, pl.ds(_DIR_ROWS[i], SEQ)],
                dst_ref=buf.at[3, pl.ds(_DIR_ROWS[i], SEQ)],
                send_sem=send.at[i], recv_sem=recv.at[i],
                device_id=(dev,), device_id_type=pl.DeviceIdType.MESH,
            )

        dir_cw = [make_dir(cwb, dcw_send, dcw_recv, i, left) for i in range(4)]
        dir_ccw = [make_dir(ccwb, dccw_send, dccw_recv, i, right)
                   for i in range(4)]

        qs = {
            (b, h): q[b * SEQ:(b + 1) * SEQ, h * D:(h + 1) * D]
            for b in range(B) for h in range(H)
        }
        state = {}

        def fold(kpart, vpart, b, head_base):
            for hh in range(HH):
                h = head_base + hh
                cs = slice(hh * D, (hh + 1) * D)
                s = lax.dot_general(
                    qs[(b, h)], kpart[:, cs], (((1,), (1,)), ((), ())),
                    preferred_element_type=jnp.float32,
                ) * SCALE
                p = jnp.exp(s)
                lj = jnp.sum(p, axis=-1, keepdims=True)
                aj = lax.dot_general(
                    p.astype(jnp.bfloat16), vpart[:, cs],
                    (((1,), (0,)), ((), ())),
                    preferred_element_type=jnp.float32,
                )
                if (b, h) not in state:
                    state[(b, h)] = (lj, aj)
                else:
                    l0, a0 = state[(b, h)]
                    state[(b, h)] = (l0 + lj, a0 + aj)

        def fold_own():
            for b in range(B):
                rs = slice(b * SEQ, (b + 1) * SEQ)
                fold(kb[rs, :HC], vb[rs, :HC], b, 0)
                fold(kb[rs, HC:], vb[rs, HC:], b, HH)

        def fold_slot(buf, slot, b, head_base):
            kpart = buf[slot, 2 * b * SEQ:(2 * b + 1) * SEQ, :]
            vpart = buf[slot, (2 * b + 1) * SEQ:(2 * b + 2) * SEQ, :]
            fold(kpart, vpart, b, head_base)

        def store_out(b, head_base):
            row = jnp.concatenate(
                [
                    state[(b, h)][1] / state[(b, h)][0]
                    for h in range(head_base, head_base + HH)
                ],
                axis=1,
            ).astype(jnp.bfloat16)
            o_ref[b * SEQ:(b + 1) * SEQ, head_base * D:(head_base + HH) * D] = row

        if _MODE == "compute_only":
            for _ in range(N_DEV):
                fold_own()
        else:
            for s in range(NSUB):
                hop1_cw[s].start()
                hop1_ccw[s].start()
            with _sc("fold_own"):
                fold_own()

            with _sc("hop1_wait_fwd"):
                hop1_cw[0].wait_recv()
                fwd_cw[0].start()
                hop1_ccw[0].wait_recv()
                fwd_ccw[0].start()
                if _MODE != "comm_only":
                    fold_slot(cwb, 1, 0, 0)
                    fold_slot(ccwb, 1, 0, HH)
                hop1_cw[1].wait_recv()
                fwd_cw[1].start()
                hop1_ccw[1].wait_recv()
                fwd_ccw[1].start()
                for i in range(4):
                    dir_cw[i].start()
                    dir_ccw[i].start()

            if _MODE != "comm_only":
                with _sc("fold_hop1"):
                    fold_slot(cwb, 1, 1, 0)
                    fold_slot(ccwb, 1, 1, HH)

            with _sc("diag"):
                for s in range(NSUB):
                    fwd_cw[s].wait_recv()
                    fwd_ccw[s].wait_recv()
                    if _MODE != "comm_only":
                        fold_slot(cwb, 2, s, 0)
                        fold_slot(ccwb, 2, s, HH)

            with _sc("direct"):
                dir_cw[0].wait_recv()
                dir_cw[1].wait_recv()
                dir_ccw[0].wait_recv()
                dir_ccw[1].wait_recv()
                pend = {}
                if _MODE != "comm_only":
                    for buf, head_base in ((cwb, 0), (ccwb, HH)):
                        for b in range(B):
                            kpart = buf[3, 2 * b * SEQ:(2 * b + 1) * SEQ, :]
                            for hh in range(HH):
                                h = head_base + hh
                                s = lax.dot_general(
                                    qs[(b, h)],
                                    kpart[:, hh * D:(hh + 1) * D],
                                    (((1,), (1,)), ((), ())),
                                    preferred_element_type=jnp.float32,
                                ) * SCALE
                                p = jnp.exp(s)
                                pend[(b, h)] = (
                                    p.astype(jnp.bfloat16),
                                    jnp.sum(p, axis=-1, keepdims=True),
                                )

                def finish(buf, b, head_base):
                    vpart = buf[3, (2 * b + 1) * SEQ:(2 * b + 2) * SEQ, :]
                    for hh in range(HH):
                        h = head_base + hh
                        pbf, lj = pend[(b, h)]
                        aj = lax.dot_general(
                            pbf, vpart[:, hh * D:(hh + 1) * D],
                            (((1,), (0,)), ((), ())),
                            preferred_element_type=jnp.float32,
                        )
                        l0, a0 = state[(b, h)]
                        state[(b, h)] = (l0 + lj, a0 + aj)

                for b in range(B):
                    dir_cw[2 + b].wait_recv()
                    dir_ccw[2 + b].wait_recv()
                    if _MODE != "comm_only":
                        finish(cwb, b, 0)
                        store_out(b, 0)
                        finish(ccwb, b, HH)
                        store_out(b, HH)

            with _sc("drain"):
                for s in range(NSUB):
                    for r in (hop1_cw[s], fwd_cw[s],
                              hop1_ccw[s], fwd_ccw[s]):
                        r.wait_send()
                for i in range(4):
                    dir_cw[i].wait_send()
                    dir_ccw[i].wait_send()

        if _MODE == "comm_only":
            o_ref[...] = kb + vb + q
        elif _MODE == "compute_only":
            for b in range(B):
                store_out(b, 0)
                store_out(b, HH)

    comm = pltpu.VMEM((4, NSUB * SUBR, HC), jnp.bfloat16)
    sems = pltpu.SemaphoreType.DMA((2, NSUB))
    dsems = pltpu.SemaphoreType.DMA((4,))
    return pl.pallas_call(
        body,
        out_shape=jax.ShapeDtypeStruct((ROWS, COLS), jnp.bfloat16),
        in_specs=[
            pl.BlockSpec(memory_space=pltpu.VMEM),
            pl.BlockSpec(memory_space=pltpu.VMEM),
            pl.BlockSpec(memory_space=pltpu.VMEM),
        ],
        out_specs=pl.BlockSpec(memory_space=pltpu.VMEM),
        scratch_shapes=[comm, comm, sems, sems, sems, sems,
                        dsems, dsems, dsems, dsems],
        compiler_params=pltpu.CompilerParams(collective_id=0),
    )(Q2, K2, V2)


def kernel(Q, K, V):
    Q2 = Q.reshape(ROWS, COLS).astype(jnp.bfloat16)
    K2 = K.reshape(ROWS, COLS).astype(jnp.bfloat16)
    V2 = V.reshape(ROWS, COLS).astype(jnp.bfloat16)
    O2 = _fused_ag_attention(Q2, K2, V2)
    return O2.reshape(B, SEQ, H, D)
